# baseline (speedup 1.0000x reference)
import functools

import jax
import jax.numpy as jnp
from jax import lax
from jax.experimental import pallas as pl
from jax.experimental.pallas import tpu as pltpu
from jax.experimental.pallas import tpu_sc as plsc

N_NODES = 10000
N_EDGES = 320000
CHUNK = 128
N_TILES = 32
CPT_SC0 = 80
CPT_SC1 = 80
PHASES_SC0 = (32, 32, 16)
PHASES_SC1 = (32, 32, 16)
E_PAD = 16 * (CPT_SC0 + CPT_SC1) * CHUNK
ACC_ROWS = 10112
ROW_BLK = 1000


def _make_seg_sum(d):
    mesh = plsc.VectorSubcoreMesh(core_axis_name="c", subcore_axis_name="s")
    d16 = d // 16
    nbuf = 2


    @functools.partial(
        pl.kernel,
        out_type=jax.ShapeDtypeStruct((2, N_NODES, d), jnp.float32),
        mesh=mesh,
        scratch_types=[
            pltpu.VMEM((32, CHUNK), jnp.int32),
            pltpu.VMEM((32, CHUNK), jnp.int32),
            [pltpu.VMEM((CHUNK, d), jnp.float32) for _ in range(nbuf)],
            pltpu.VMEM_SHARED((ACC_ROWS, d), jnp.float32),
            [pltpu.SemaphoreType.DMA for _ in range(nbuf)],
        ],
    )
    def seg(y_hbm, src_hbm, dst_hbm, out_hbm, src_v, dst_v, rows_v, acc_sh, sems):
        cid = lax.axis_index("c")
        sid = lax.axis_index("s")

        def zbody(i, _):
            r = i // d16
            k = i - r * d16
            rows_v[0][r, pl.ds(k * 16, 16)] = jnp.zeros((16,), jnp.float32)
            return 0

        with jax.named_scope("zz_zero"):
            lax.fori_loop(0, CHUNK * d16, zbody, 0)
            spt = ACC_ROWS // 16
            for t in range(spt // CHUNK):
                pltpu.sync_copy(
                    rows_v[0], acc_sh.at[pl.ds(sid * spt + t * CHUNK, CHUNK)]
                )
            rem = spt % CHUNK
            if rem:
                pltpu.sync_copy(
                    rows_v[0].at[pl.ds(0, rem)],
                    acc_sh.at[pl.ds(sid * spt + spt - rem, rem)],
                )
            plsc.subcore_barrier()

        def gather(j, b):
            return pltpu.make_async_copy(y_hbm.at[src_v.at[j]], rows_v[b], sems[b])

        def run_phase(base, cpp):
            pltpu.sync_copy(src_hbm.at[pl.ds(base, cpp)], src_v.at[pl.ds(0, cpp)])
            pltpu.sync_copy(dst_hbm.at[pl.ds(base, cpp)], dst_v.at[pl.ds(0, cpp)])

            for i in range(nbuf - 1):
                gather(i, i).start()

            def body(m, _):
                for i in range(nbuf):
                    j = m * nbuf + i

                    @pl.when(j + nbuf - 1 < cpp)
                    def _():
                        gather(j + nbuf - 1, (i + nbuf - 1) % nbuf).start()

                    gather(j, i).wait()
                    pltpu.sync_copy(rows_v[i], acc_sh.at[dst_v.at[j]], add=True)
                return 0

            lax.fori_loop(0, cpp // nbuf, body, 0)

        with jax.named_scope("zz_edges"):
            @pl.when(cid == 0)
            def _():
                off = 0
                for cpp in PHASES_SC0:
                    run_phase(sid * CPT_SC0 + off, cpp)
                    off += cpp

            @pl.when(cid == 1)
            def _():
                off = 0
                for cpp in PHASES_SC1:
                    run_phase(16 * CPT_SC0 + sid * CPT_SC1 + off, cpp)
                    off += cpp

            plsc.subcore_barrier()

        @pl.when(sid < 15)
        def _():
            pltpu.sync_copy(
                acc_sh.at[pl.ds(sid * 624, 624)],
                out_hbm.at[cid, pl.ds(sid * 624, 624)],
            )

        @pl.when(sid == 15)
        def _():
            pltpu.sync_copy(
                acc_sh.at[pl.ds(15 * 624, 640)],
                out_hbm.at[cid, pl.ds(15 * 624, 640)],
            )

    return seg


def _tc_first(x, w_rel, w_root, b):
    dout = w_rel.shape[1]

    def body(x_ref, wr_ref, wt_ref, b_ref, y_ref, r_ref):
        xb = x_ref[...]
        y_ref[...] = jnp.dot(xb, wr_ref[...], preferred_element_type=jnp.float32)
        r_ref[...] = (
            jnp.dot(xb, wt_ref[...], preferred_element_type=jnp.float32)
            + b_ref[...]
        )

    grid = (N_NODES // ROW_BLK,)
    blk = lambda i: (i, 0)
    full = lambda i: (0, 0)
    return pl.pallas_call(
        body,
        grid=grid,
        in_specs=[
            pl.BlockSpec((ROW_BLK, x.shape[1]), blk),
            pl.BlockSpec(w_rel.shape, full),
            pl.BlockSpec(w_root.shape, full),
            pl.BlockSpec((1, dout), full),
        ],
        out_specs=[
            pl.BlockSpec((ROW_BLK, dout), blk),
            pl.BlockSpec((ROW_BLK, dout), blk),
        ],
        out_shape=[
            jax.ShapeDtypeStruct((N_NODES, dout), jnp.float32),
            jax.ShapeDtypeStruct((N_NODES, dout), jnp.float32),
        ],
    )(x, w_rel, w_root, b.reshape(1, dout))


def _tc_mid(parts, r_prev, w_rel, w_root, b):
    din = r_prev.shape[1]
    dout = w_rel.shape[1]

    def body(p_ref, rp_ref, wr_ref, wt_ref, b_ref, y_ref, r_ref):
        h = p_ref[0] + p_ref[1] + rp_ref[...]
        h = h * jax.nn.sigmoid(h)
        y_ref[...] = jnp.dot(h, wr_ref[...], preferred_element_type=jnp.float32)
        r_ref[...] = (
            jnp.dot(h, wt_ref[...], preferred_element_type=jnp.float32)
            + b_ref[...]
        )

    grid = (N_NODES // ROW_BLK,)
    return pl.pallas_call(
        body,
        grid=grid,
        in_specs=[
            pl.BlockSpec((2, ROW_BLK, din), lambda i: (0, i, 0)),
            pl.BlockSpec((ROW_BLK, din), lambda i: (i, 0)),
            pl.BlockSpec(w_rel.shape, lambda i: (0, 0)),
            pl.BlockSpec(w_root.shape, lambda i: (0, 0)),
            pl.BlockSpec((1, dout), lambda i: (0, 0)),
        ],
        out_specs=[
            pl.BlockSpec((ROW_BLK, dout), lambda i: (i, 0)),
            pl.BlockSpec((ROW_BLK, dout), lambda i: (i, 0)),
        ],
        out_shape=[
            jax.ShapeDtypeStruct((N_NODES, dout), jnp.float32),
            jax.ShapeDtypeStruct((N_NODES, dout), jnp.float32),
        ],
    )(parts, r_prev, w_rel, w_root, b.reshape(1, dout))


def _tc_mid_h(parts, r_prev, w_root, b):
    din = r_prev.shape[1]
    dout = w_root.shape[1]

    def body(p_ref, rp_ref, wt_ref, b_ref, h_ref, r_ref):
        h = p_ref[0] + p_ref[1] + rp_ref[...]
        h = h * jax.nn.sigmoid(h)
        h_ref[...] = h
        r_ref[...] = (
            jnp.dot(h, wt_ref[...], preferred_element_type=jnp.float32)
            + b_ref[...]
        )

    grid = (N_NODES // ROW_BLK,)
    return pl.pallas_call(
        body,
        grid=grid,
        in_specs=[
            pl.BlockSpec((2, ROW_BLK, din), lambda i: (0, i, 0)),
            pl.BlockSpec((ROW_BLK, din), lambda i: (i, 0)),
            pl.BlockSpec(w_root.shape, lambda i: (0, 0)),
            pl.BlockSpec((1, dout), lambda i: (0, 0)),
        ],
        out_specs=[
            pl.BlockSpec((ROW_BLK, din), lambda i: (i, 0)),
            pl.BlockSpec((ROW_BLK, dout), lambda i: (i, 0)),
        ],
        out_shape=[
            jax.ShapeDtypeStruct((N_NODES, din), jnp.float32),
            jax.ShapeDtypeStruct((N_NODES, dout), jnp.float32),
        ],
    )(parts, r_prev, w_root, b.reshape(1, dout))


def _tc_final(parts, r_prev, w_rel):
    din = w_rel.shape[0]
    d = r_prev.shape[1]

    def body(p_ref, rp_ref, wr_ref, o_ref):
        g = p_ref[0] + p_ref[1]
        h = (
            jnp.dot(g, wr_ref[...], preferred_element_type=jnp.float32)
            + rp_ref[...]
        )
        m = jnp.max(h, axis=1, keepdims=True)
        s = h - m
        o_ref[...] = s - jnp.log(jnp.sum(jnp.exp(s), axis=1, keepdims=True))

    grid = (N_NODES // ROW_BLK,)
    return pl.pallas_call(
        body,
        grid=grid,
        in_specs=[
            pl.BlockSpec((2, ROW_BLK, din), lambda i: (0, i, 0)),
            pl.BlockSpec((ROW_BLK, d), lambda i: (i, 0)),
            pl.BlockSpec(w_rel.shape, lambda i: (0, 0)),
        ],
        out_specs=pl.BlockSpec((ROW_BLK, d), lambda i: (i, 0)),
        out_shape=jax.ShapeDtypeStruct((N_NODES, d), jnp.float32),
    )(parts, r_prev, w_rel)


_seg_sum_128 = _make_seg_sum(128)


def kernel(x, edge_index, W_rel1, b_rel1, W_root1, W_rel2, b_rel2, W_root2,
           W_rel3, b_rel3, W_root3):
    pad = E_PAD - N_EDGES
    src = jnp.concatenate(
        [edge_index[0].astype(jnp.int32),
         jnp.arange(pad, dtype=jnp.int32) % N_NODES]
    ).reshape(E_PAD // CHUNK, CHUNK)
    dst = jnp.concatenate(
        [edge_index[1].astype(jnp.int32),
         N_NODES + jnp.arange(pad, dtype=jnp.int32) % (ACC_ROWS - N_NODES)]
    ).reshape(E_PAD // CHUNK, CHUNK)

    y1, r1 = _tc_first(x, W_rel1, W_root1, b_rel1)
    p1 = _seg_sum_128(y1, src, dst)
    y2, r2 = _tc_mid(p1, r1, W_rel2, W_root2, b_rel2)
    p2 = _seg_sum_128(y2, src, dst)
    h2, r3 = _tc_mid_h(p2, r2, W_root3, b_rel3)
    p3 = _seg_sum_128(h2, src, dst)
    return _tc_final(p3, r3, W_rel3)


# speedup vs baseline: 1.0994x; 1.0994x over previous
import functools

import jax
import jax.numpy as jnp
from jax import lax
from jax.experimental import pallas as pl
from jax.experimental.pallas import tpu as pltpu
from jax.experimental.pallas import tpu_sc as plsc

N_NODES = 10000
N_EDGES = 320000
CHUNK = 64
N_TILES = 32
CPT_SC0 = 160
CPT_SC1 = 160
PHASES_SC0 = (64, 64, 32)
PHASES_SC1 = (64, 64, 32)
E_PAD = 16 * (CPT_SC0 + CPT_SC1) * CHUNK
ACC_ROWS = 10112
ROW_BLK = 1000


def _make_seg_sum(d):
    mesh = plsc.VectorSubcoreMesh(core_axis_name="c", subcore_axis_name="s")
    d16 = d // 16
    nbuf = 4


    @functools.partial(
        pl.kernel,
        out_type=jax.ShapeDtypeStruct((2, N_NODES, d), jnp.float32),
        mesh=mesh,
        scratch_types=[
            pltpu.VMEM((64, CHUNK), jnp.int32),
            pltpu.VMEM((64, CHUNK), jnp.int32),
            [pltpu.VMEM((CHUNK, d), jnp.float32) for _ in range(nbuf)],
            pltpu.VMEM_SHARED((ACC_ROWS, d), jnp.float32),
            [pltpu.SemaphoreType.DMA for _ in range(nbuf)],
        ],
    )
    def seg(y_hbm, src_hbm, dst_hbm, out_hbm, src_v, dst_v, rows_v, acc_sh, sems):
        cid = lax.axis_index("c")
        sid = lax.axis_index("s")

        def zbody(i, _):
            r = i // d16
            k = i - r * d16
            rows_v[0][r, pl.ds(k * 16, 16)] = jnp.zeros((16,), jnp.float32)
            return 0

        with jax.named_scope("zz_zero"):
            lax.fori_loop(0, CHUNK * d16, zbody, 0)
            spt = ACC_ROWS // 16
            for t in range(spt // CHUNK):
                pltpu.sync_copy(
                    rows_v[0], acc_sh.at[pl.ds(sid * spt + t * CHUNK, CHUNK)]
                )
            rem = spt % CHUNK
            if rem:
                pltpu.sync_copy(
                    rows_v[0].at[pl.ds(0, rem)],
                    acc_sh.at[pl.ds(sid * spt + spt - rem, rem)],
                )
            plsc.subcore_barrier()

        def gather(j, b):
            return pltpu.make_async_copy(y_hbm.at[src_v.at[j]], rows_v[b], sems[b])

        def run_phase(base, cpp):
            pltpu.sync_copy(src_hbm.at[pl.ds(base, cpp)], src_v.at[pl.ds(0, cpp)])
            pltpu.sync_copy(dst_hbm.at[pl.ds(base, cpp)], dst_v.at[pl.ds(0, cpp)])

            for i in range(nbuf - 1):
                gather(i, i).start()

            def body(m, _):
                for i in range(nbuf):
                    j = m * nbuf + i

                    @pl.when(j + nbuf - 1 < cpp)
                    def _():
                        gather(j + nbuf - 1, (i + nbuf - 1) % nbuf).start()

                    gather(j, i).wait()
                    pltpu.sync_copy(rows_v[i], acc_sh.at[dst_v.at[j]], add=True)
                return 0

            lax.fori_loop(0, cpp // nbuf, body, 0)

        with jax.named_scope("zz_edges"):
            @pl.when(cid == 0)
            def _():
                off = 0
                for cpp in PHASES_SC0:
                    run_phase(sid * CPT_SC0 + off, cpp)
                    off += cpp

            @pl.when(cid == 1)
            def _():
                off = 0
                for cpp in PHASES_SC1:
                    run_phase(16 * CPT_SC0 + sid * CPT_SC1 + off, cpp)
                    off += cpp

            plsc.subcore_barrier()

        @pl.when(sid < 15)
        def _():
            pltpu.sync_copy(
                acc_sh.at[pl.ds(sid * 624, 624)],
                out_hbm.at[cid, pl.ds(sid * 624, 624)],
            )

        @pl.when(sid == 15)
        def _():
            pltpu.sync_copy(
                acc_sh.at[pl.ds(15 * 624, 640)],
                out_hbm.at[cid, pl.ds(15 * 624, 640)],
            )

    return seg


def _tc_first(x, w_rel, w_root, b):
    dout = w_rel.shape[1]

    def body(x_ref, wr_ref, wt_ref, b_ref, y_ref, r_ref):
        xb = x_ref[...]
        y_ref[...] = jnp.dot(xb, wr_ref[...], preferred_element_type=jnp.float32)
        r_ref[...] = (
            jnp.dot(xb, wt_ref[...], preferred_element_type=jnp.float32)
            + b_ref[...]
        )

    grid = (N_NODES // ROW_BLK,)
    blk = lambda i: (i, 0)
    full = lambda i: (0, 0)
    return pl.pallas_call(
        body,
        grid=grid,
        in_specs=[
            pl.BlockSpec((ROW_BLK, x.shape[1]), blk),
            pl.BlockSpec(w_rel.shape, full),
            pl.BlockSpec(w_root.shape, full),
            pl.BlockSpec((1, dout), full),
        ],
        out_specs=[
            pl.BlockSpec((ROW_BLK, dout), blk),
            pl.BlockSpec((ROW_BLK, dout), blk),
        ],
        out_shape=[
            jax.ShapeDtypeStruct((N_NODES, dout), jnp.float32),
            jax.ShapeDtypeStruct((N_NODES, dout), jnp.float32),
        ],
    )(x, w_rel, w_root, b.reshape(1, dout))


def _tc_mid(parts, r_prev, w_rel, w_root, b):
    din = r_prev.shape[1]
    dout = w_rel.shape[1]

    def body(p_ref, rp_ref, wr_ref, wt_ref, b_ref, y_ref, r_ref):
        h = p_ref[0] + p_ref[1] + rp_ref[...]
        h = h * jax.nn.sigmoid(h)
        y_ref[...] = jnp.dot(h, wr_ref[...], preferred_element_type=jnp.float32)
        r_ref[...] = (
            jnp.dot(h, wt_ref[...], preferred_element_type=jnp.float32)
            + b_ref[...]
        )

    grid = (N_NODES // ROW_BLK,)
    return pl.pallas_call(
        body,
        grid=grid,
        in_specs=[
            pl.BlockSpec((2, ROW_BLK, din), lambda i: (0, i, 0)),
            pl.BlockSpec((ROW_BLK, din), lambda i: (i, 0)),
            pl.BlockSpec(w_rel.shape, lambda i: (0, 0)),
            pl.BlockSpec(w_root.shape, lambda i: (0, 0)),
            pl.BlockSpec((1, dout), lambda i: (0, 0)),
        ],
        out_specs=[
            pl.BlockSpec((ROW_BLK, dout), lambda i: (i, 0)),
            pl.BlockSpec((ROW_BLK, dout), lambda i: (i, 0)),
        ],
        out_shape=[
            jax.ShapeDtypeStruct((N_NODES, dout), jnp.float32),
            jax.ShapeDtypeStruct((N_NODES, dout), jnp.float32),
        ],
    )(parts, r_prev, w_rel, w_root, b.reshape(1, dout))


def _tc_mid_h(parts, r_prev, w_root, b):
    din = r_prev.shape[1]
    dout = w_root.shape[1]

    def body(p_ref, rp_ref, wt_ref, b_ref, h_ref, r_ref):
        h = p_ref[0] + p_ref[1] + rp_ref[...]
        h = h * jax.nn.sigmoid(h)
        h_ref[...] = h
        r_ref[...] = (
            jnp.dot(h, wt_ref[...], preferred_element_type=jnp.float32)
            + b_ref[...]
        )

    grid = (N_NODES // ROW_BLK,)
    return pl.pallas_call(
        body,
        grid=grid,
        in_specs=[
            pl.BlockSpec((2, ROW_BLK, din), lambda i: (0, i, 0)),
            pl.BlockSpec((ROW_BLK, din), lambda i: (i, 0)),
            pl.BlockSpec(w_root.shape, lambda i: (0, 0)),
            pl.BlockSpec((1, dout), lambda i: (0, 0)),
        ],
        out_specs=[
            pl.BlockSpec((ROW_BLK, din), lambda i: (i, 0)),
            pl.BlockSpec((ROW_BLK, dout), lambda i: (i, 0)),
        ],
        out_shape=[
            jax.ShapeDtypeStruct((N_NODES, din), jnp.float32),
            jax.ShapeDtypeStruct((N_NODES, dout), jnp.float32),
        ],
    )(parts, r_prev, w_root, b.reshape(1, dout))


def _tc_final(parts, r_prev, w_rel):
    din = w_rel.shape[0]
    d = r_prev.shape[1]

    def body(p_ref, rp_ref, wr_ref, o_ref):
        g = p_ref[0] + p_ref[1]
        h = (
            jnp.dot(g, wr_ref[...], preferred_element_type=jnp.float32)
            + rp_ref[...]
        )
        m = jnp.max(h, axis=1, keepdims=True)
        s = h - m
        o_ref[...] = s - jnp.log(jnp.sum(jnp.exp(s), axis=1, keepdims=True))

    grid = (N_NODES // ROW_BLK,)
    return pl.pallas_call(
        body,
        grid=grid,
        in_specs=[
            pl.BlockSpec((2, ROW_BLK, din), lambda i: (0, i, 0)),
            pl.BlockSpec((ROW_BLK, d), lambda i: (i, 0)),
            pl.BlockSpec(w_rel.shape, lambda i: (0, 0)),
        ],
        out_specs=pl.BlockSpec((ROW_BLK, d), lambda i: (i, 0)),
        out_shape=jax.ShapeDtypeStruct((N_NODES, d), jnp.float32),
    )(parts, r_prev, w_rel)


_seg_sum_128 = _make_seg_sum(128)


def kernel(x, edge_index, W_rel1, b_rel1, W_root1, W_rel2, b_rel2, W_root2,
           W_rel3, b_rel3, W_root3):
    pad = E_PAD - N_EDGES
    src = jnp.concatenate(
        [edge_index[0].astype(jnp.int32),
         jnp.arange(pad, dtype=jnp.int32) % N_NODES]
    ).reshape(E_PAD // CHUNK, CHUNK)
    dst = jnp.concatenate(
        [edge_index[1].astype(jnp.int32),
         N_NODES + jnp.arange(pad, dtype=jnp.int32) % (ACC_ROWS - N_NODES)]
    ).reshape(E_PAD // CHUNK, CHUNK)

    y1, r1 = _tc_first(x, W_rel1, W_root1, b_rel1)
    p1 = _seg_sum_128(y1, src, dst)
    y2, r2 = _tc_mid(p1, r1, W_rel2, W_root2, b_rel2)
    p2 = _seg_sum_128(y2, src, dst)
    h2, r3 = _tc_mid_h(p2, r2, W_root3, b_rel3)
    p3 = _seg_sum_128(h2, src, dst)
    return _tc_final(p3, r3, W_rel3)


# speedup vs baseline: 1.1249x; 1.0232x over previous
import functools

import jax
import jax.numpy as jnp
from jax import lax
from jax.experimental import pallas as pl
from jax.experimental.pallas import tpu as pltpu
from jax.experimental.pallas import tpu_sc as plsc

N_NODES = 10000
N_EDGES = 320000
CHUNK = 64
N_TILES = 32
CPT_SC0 = 160
CPT_SC1 = 160
PHASES_SC0 = (64, 64, 32)
PHASES_SC1 = (64, 64, 32)
E_PAD = 16 * (CPT_SC0 + CPT_SC1) * CHUNK
ACC_ROWS = 10112
ROW_BLK = 2000


def _make_seg_sum(d):
    mesh = plsc.VectorSubcoreMesh(core_axis_name="c", subcore_axis_name="s")
    d16 = d // 16
    nbuf = 4


    @functools.partial(
        pl.kernel,
        out_type=jax.ShapeDtypeStruct((2, N_NODES, d), jnp.float32),
        mesh=mesh,
        scratch_types=[
            pltpu.VMEM((64, CHUNK), jnp.int32),
            pltpu.VMEM((64, CHUNK), jnp.int32),
            [pltpu.VMEM((CHUNK, d), jnp.float32) for _ in range(nbuf)],
            pltpu.VMEM_SHARED((ACC_ROWS, d), jnp.float32),
            [pltpu.SemaphoreType.DMA for _ in range(nbuf)],
        ],
    )
    def seg(y_hbm, src_hbm, dst_hbm, out_hbm, src_v, dst_v, rows_v, acc_sh, sems):
        cid = lax.axis_index("c")
        sid = lax.axis_index("s")

        def zbody(i, _):
            r = i // d16
            k = i - r * d16
            rows_v[0][r, pl.ds(k * 16, 16)] = jnp.zeros((16,), jnp.float32)
            return 0

        with jax.named_scope("zz_zero"):
            lax.fori_loop(0, CHUNK * d16, zbody, 0)
            spt = ACC_ROWS // 16
            for t in range(spt // CHUNK):
                pltpu.sync_copy(
                    rows_v[0], acc_sh.at[pl.ds(sid * spt + t * CHUNK, CHUNK)]
                )
            rem = spt % CHUNK
            if rem:
                pltpu.sync_copy(
                    rows_v[0].at[pl.ds(0, rem)],
                    acc_sh.at[pl.ds(sid * spt + spt - rem, rem)],
                )
            plsc.subcore_barrier()

        def gather(j, b):
            return pltpu.make_async_copy(y_hbm.at[src_v.at[j]], rows_v[b], sems[b])

        def run_phase(base, cpp):
            pltpu.sync_copy(src_hbm.at[pl.ds(base, cpp)], src_v.at[pl.ds(0, cpp)])
            pltpu.sync_copy(dst_hbm.at[pl.ds(base, cpp)], dst_v.at[pl.ds(0, cpp)])

            for i in range(nbuf - 1):
                gather(i, i).start()

            def body(m, _):
                for i in range(nbuf):
                    j = m * nbuf + i

                    @pl.when(j + nbuf - 1 < cpp)
                    def _():
                        gather(j + nbuf - 1, (i + nbuf - 1) % nbuf).start()

                    gather(j, i).wait()
                    pltpu.sync_copy(rows_v[i], acc_sh.at[dst_v.at[j]], add=True)
                return 0

            lax.fori_loop(0, cpp // nbuf, body, 0)

        with jax.named_scope("zz_edges"):
            @pl.when(cid == 0)
            def _():
                off = 0
                for cpp in PHASES_SC0:
                    run_phase(sid * CPT_SC0 + off, cpp)
                    off += cpp

            @pl.when(cid == 1)
            def _():
                off = 0
                for cpp in PHASES_SC1:
                    run_phase(16 * CPT_SC0 + sid * CPT_SC1 + off, cpp)
                    off += cpp

            plsc.subcore_barrier()

        @pl.when(sid < 15)
        def _():
            pltpu.sync_copy(
                acc_sh.at[pl.ds(sid * 624, 624)],
                out_hbm.at[cid, pl.ds(sid * 624, 624)],
            )

        @pl.when(sid == 15)
        def _():
            pltpu.sync_copy(
                acc_sh.at[pl.ds(15 * 624, 640)],
                out_hbm.at[cid, pl.ds(15 * 624, 640)],
            )

    return seg


def _tc_first(x, w_rel, w_root, b):
    dout = w_rel.shape[1]

    def body(x_ref, wr_ref, wt_ref, b_ref, y_ref, r_ref):
        xb = x_ref[...]
        y_ref[...] = jnp.dot(xb, wr_ref[...], preferred_element_type=jnp.float32)
        r_ref[...] = (
            jnp.dot(xb, wt_ref[...], preferred_element_type=jnp.float32)
            + b_ref[...]
        )

    grid = (N_NODES // ROW_BLK,)
    blk = lambda i: (i, 0)
    full = lambda i: (0, 0)
    return pl.pallas_call(
        body,
        grid=grid,
        in_specs=[
            pl.BlockSpec((ROW_BLK, x.shape[1]), blk),
            pl.BlockSpec(w_rel.shape, full),
            pl.BlockSpec(w_root.shape, full),
            pl.BlockSpec((1, dout), full),
        ],
        out_specs=[
            pl.BlockSpec((ROW_BLK, dout), blk),
            pl.BlockSpec((ROW_BLK, dout), blk),
        ],
        out_shape=[
            jax.ShapeDtypeStruct((N_NODES, dout), jnp.float32),
            jax.ShapeDtypeStruct((N_NODES, dout), jnp.float32),
        ],
    )(x, w_rel, w_root, b.reshape(1, dout))


def _tc_mid(parts, r_prev, w_rel, w_root, b):
    din = r_prev.shape[1]
    dout = w_rel.shape[1]

    def body(p_ref, rp_ref, wr_ref, wt_ref, b_ref, y_ref, r_ref):
        h = p_ref[0] + p_ref[1] + rp_ref[...]
        h = h * jax.nn.sigmoid(h)
        y_ref[...] = jnp.dot(h, wr_ref[...], preferred_element_type=jnp.float32)
        r_ref[...] = (
            jnp.dot(h, wt_ref[...], preferred_element_type=jnp.float32)
            + b_ref[...]
        )

    grid = (N_NODES // ROW_BLK,)
    return pl.pallas_call(
        body,
        grid=grid,
        in_specs=[
            pl.BlockSpec((2, ROW_BLK, din), lambda i: (0, i, 0)),
            pl.BlockSpec((ROW_BLK, din), lambda i: (i, 0)),
            pl.BlockSpec(w_rel.shape, lambda i: (0, 0)),
            pl.BlockSpec(w_root.shape, lambda i: (0, 0)),
            pl.BlockSpec((1, dout), lambda i: (0, 0)),
        ],
        out_specs=[
            pl.BlockSpec((ROW_BLK, dout), lambda i: (i, 0)),
            pl.BlockSpec((ROW_BLK, dout), lambda i: (i, 0)),
        ],
        out_shape=[
            jax.ShapeDtypeStruct((N_NODES, dout), jnp.float32),
            jax.ShapeDtypeStruct((N_NODES, dout), jnp.float32),
        ],
    )(parts, r_prev, w_rel, w_root, b.reshape(1, dout))


def _tc_mid_h(parts, r_prev, w_root, b):
    din = r_prev.shape[1]
    dout = w_root.shape[1]

    def body(p_ref, rp_ref, wt_ref, b_ref, h_ref, r_ref):
        h = p_ref[0] + p_ref[1] + rp_ref[...]
        h = h * jax.nn.sigmoid(h)
        h_ref[...] = h
        r_ref[...] = (
            jnp.dot(h, wt_ref[...], preferred_element_type=jnp.float32)
            + b_ref[...]
        )

    grid = (N_NODES // ROW_BLK,)
    return pl.pallas_call(
        body,
        grid=grid,
        in_specs=[
            pl.BlockSpec((2, ROW_BLK, din), lambda i: (0, i, 0)),
            pl.BlockSpec((ROW_BLK, din), lambda i: (i, 0)),
            pl.BlockSpec(w_root.shape, lambda i: (0, 0)),
            pl.BlockSpec((1, dout), lambda i: (0, 0)),
        ],
        out_specs=[
            pl.BlockSpec((ROW_BLK, din), lambda i: (i, 0)),
            pl.BlockSpec((ROW_BLK, dout), lambda i: (i, 0)),
        ],
        out_shape=[
            jax.ShapeDtypeStruct((N_NODES, din), jnp.float32),
            jax.ShapeDtypeStruct((N_NODES, dout), jnp.float32),
        ],
    )(parts, r_prev, w_root, b.reshape(1, dout))


def _tc_final(parts, r_prev, w_rel):
    din = w_rel.shape[0]
    d = r_prev.shape[1]

    def body(p_ref, rp_ref, wr_ref, o_ref):
        g = p_ref[0] + p_ref[1]
        h = (
            jnp.dot(g, wr_ref[...], preferred_element_type=jnp.float32)
            + rp_ref[...]
        )
        m = jnp.max(h, axis=1, keepdims=True)
        s = h - m
        o_ref[...] = s - jnp.log(jnp.sum(jnp.exp(s), axis=1, keepdims=True))

    grid = (N_NODES // ROW_BLK,)
    return pl.pallas_call(
        body,
        grid=grid,
        in_specs=[
            pl.BlockSpec((2, ROW_BLK, din), lambda i: (0, i, 0)),
            pl.BlockSpec((ROW_BLK, d), lambda i: (i, 0)),
            pl.BlockSpec(w_rel.shape, lambda i: (0, 0)),
        ],
        out_specs=pl.BlockSpec((ROW_BLK, d), lambda i: (i, 0)),
        out_shape=jax.ShapeDtypeStruct((N_NODES, d), jnp.float32),
    )(parts, r_prev, w_rel)


_seg_sum_128 = _make_seg_sum(128)


def kernel(x, edge_index, W_rel1, b_rel1, W_root1, W_rel2, b_rel2, W_root2,
           W_rel3, b_rel3, W_root3):
    pad = E_PAD - N_EDGES
    src = jnp.concatenate(
        [edge_index[0].astype(jnp.int32),
         jnp.arange(pad, dtype=jnp.int32) % N_NODES]
    ).reshape(E_PAD // CHUNK, CHUNK)
    dst = jnp.concatenate(
        [edge_index[1].astype(jnp.int32),
         N_NODES + jnp.arange(pad, dtype=jnp.int32) % (ACC_ROWS - N_NODES)]
    ).reshape(E_PAD // CHUNK, CHUNK)

    y1, r1 = _tc_first(x, W_rel1, W_root1, b_rel1)
    p1 = _seg_sum_128(y1, src, dst)
    y2, r2 = _tc_mid(p1, r1, W_rel2, W_root2, b_rel2)
    p2 = _seg_sum_128(y2, src, dst)
    h2, r3 = _tc_mid_h(p2, r2, W_root3, b_rel3)
    p3 = _seg_sum_128(h2, src, dst)
    return _tc_final(p3, r3, W_rel3)


# speedup vs baseline: 1.1471x; 1.0198x over previous
import functools

import jax
import jax.numpy as jnp
from jax import lax
from jax.experimental import pallas as pl
from jax.experimental.pallas import tpu as pltpu
from jax.experimental.pallas import tpu_sc as plsc

N_NODES = 10000
N_EDGES = 320000
CHUNK = 64
N_TILES = 32
CPT_SC0 = 160
CPT_SC1 = 160
PHASES_SC0 = (64, 64, 32)
PHASES_SC1 = (64, 64, 32)
E_PAD = 16 * (CPT_SC0 + CPT_SC1) * CHUNK
ACC_ROWS = 10112
ROW_BLK = 5000


def _make_seg_sum(d):
    mesh = plsc.VectorSubcoreMesh(core_axis_name="c", subcore_axis_name="s")
    d16 = d // 16
    nbuf = 4


    @functools.partial(
        pl.kernel,
        out_type=jax.ShapeDtypeStruct((2, N_NODES, d), jnp.float32),
        mesh=mesh,
        scratch_types=[
            pltpu.VMEM((64, CHUNK), jnp.int32),
            pltpu.VMEM((64, CHUNK), jnp.int32),
            [pltpu.VMEM((CHUNK, d), jnp.float32) for _ in range(nbuf)],
            pltpu.VMEM_SHARED((ACC_ROWS, d), jnp.float32),
            [pltpu.SemaphoreType.DMA for _ in range(nbuf)],
        ],
    )
    def seg(y_hbm, src_hbm, dst_hbm, out_hbm, src_v, dst_v, rows_v, acc_sh, sems):
        cid = lax.axis_index("c")
        sid = lax.axis_index("s")

        def zbody(i, _):
            r = i // d16
            k = i - r * d16
            rows_v[0][r, pl.ds(k * 16, 16)] = jnp.zeros((16,), jnp.float32)
            return 0

        with jax.named_scope("zz_zero"):
            lax.fori_loop(0, CHUNK * d16, zbody, 0)
            spt = ACC_ROWS // 16
            for t in range(spt // CHUNK):
                pltpu.sync_copy(
                    rows_v[0], acc_sh.at[pl.ds(sid * spt + t * CHUNK, CHUNK)]
                )
            rem = spt % CHUNK
            if rem:
                pltpu.sync_copy(
                    rows_v[0].at[pl.ds(0, rem)],
                    acc_sh.at[pl.ds(sid * spt + spt - rem, rem)],
                )
            plsc.subcore_barrier()

        def gather(j, b):
            return pltpu.make_async_copy(y_hbm.at[src_v.at[j]], rows_v[b], sems[b])

        def run_phase(base, cpp):
            pltpu.sync_copy(src_hbm.at[pl.ds(base, cpp)], src_v.at[pl.ds(0, cpp)])
            pltpu.sync_copy(dst_hbm.at[pl.ds(base, cpp)], dst_v.at[pl.ds(0, cpp)])

            for i in range(nbuf - 1):
                gather(i, i).start()

            def body(m, _):
                for i in range(nbuf):
                    j = m * nbuf + i

                    @pl.when(j + nbuf - 1 < cpp)
                    def _():
                        gather(j + nbuf - 1, (i + nbuf - 1) % nbuf).start()

                    gather(j, i).wait()
                    pltpu.sync_copy(rows_v[i], acc_sh.at[dst_v.at[j]], add=True)
                return 0

            lax.fori_loop(0, cpp // nbuf, body, 0)

        with jax.named_scope("zz_edges"):
            @pl.when(cid == 0)
            def _():
                off = 0
                for cpp in PHASES_SC0:
                    run_phase(sid * CPT_SC0 + off, cpp)
                    off += cpp

            @pl.when(cid == 1)
            def _():
                off = 0
                for cpp in PHASES_SC1:
                    run_phase(16 * CPT_SC0 + sid * CPT_SC1 + off, cpp)
                    off += cpp

            plsc.subcore_barrier()

        @pl.when(sid < 15)
        def _():
            pltpu.sync_copy(
                acc_sh.at[pl.ds(sid * 624, 624)],
                out_hbm.at[cid, pl.ds(sid * 624, 624)],
            )

        @pl.when(sid == 15)
        def _():
            pltpu.sync_copy(
                acc_sh.at[pl.ds(15 * 624, 640)],
                out_hbm.at[cid, pl.ds(15 * 624, 640)],
            )

    return seg


def _tc_first(x, w_rel, w_root, b):
    dout = w_rel.shape[1]

    def body(x_ref, wr_ref, wt_ref, b_ref, y_ref, r_ref):
        xb = x_ref[...]
        y_ref[...] = jnp.dot(xb, wr_ref[...], preferred_element_type=jnp.float32)
        r_ref[...] = (
            jnp.dot(xb, wt_ref[...], preferred_element_type=jnp.float32)
            + b_ref[...]
        )

    grid = (N_NODES // ROW_BLK,)
    blk = lambda i: (i, 0)
    full = lambda i: (0, 0)
    return pl.pallas_call(
        body,
        grid=grid,
        in_specs=[
            pl.BlockSpec((ROW_BLK, x.shape[1]), blk),
            pl.BlockSpec(w_rel.shape, full),
            pl.BlockSpec(w_root.shape, full),
            pl.BlockSpec((1, dout), full),
        ],
        out_specs=[
            pl.BlockSpec((ROW_BLK, dout), blk),
            pl.BlockSpec((ROW_BLK, dout), blk),
        ],
        out_shape=[
            jax.ShapeDtypeStruct((N_NODES, dout), jnp.float32),
            jax.ShapeDtypeStruct((N_NODES, dout), jnp.float32),
        ],
    )(x, w_rel, w_root, b.reshape(1, dout))


def _tc_mid(parts, r_prev, w_rel, w_root, b):
    din = r_prev.shape[1]
    dout = w_rel.shape[1]

    def body(p_ref, rp_ref, wr_ref, wt_ref, b_ref, y_ref, r_ref):
        h = p_ref[0] + p_ref[1] + rp_ref[...]
        h = h * jax.nn.sigmoid(h)
        y_ref[...] = jnp.dot(h, wr_ref[...], preferred_element_type=jnp.float32)
        r_ref[...] = (
            jnp.dot(h, wt_ref[...], preferred_element_type=jnp.float32)
            + b_ref[...]
        )

    grid = (N_NODES // ROW_BLK,)
    return pl.pallas_call(
        body,
        grid=grid,
        in_specs=[
            pl.BlockSpec((2, ROW_BLK, din), lambda i: (0, i, 0)),
            pl.BlockSpec((ROW_BLK, din), lambda i: (i, 0)),
            pl.BlockSpec(w_rel.shape, lambda i: (0, 0)),
            pl.BlockSpec(w_root.shape, lambda i: (0, 0)),
            pl.BlockSpec((1, dout), lambda i: (0, 0)),
        ],
        out_specs=[
            pl.BlockSpec((ROW_BLK, dout), lambda i: (i, 0)),
            pl.BlockSpec((ROW_BLK, dout), lambda i: (i, 0)),
        ],
        out_shape=[
            jax.ShapeDtypeStruct((N_NODES, dout), jnp.float32),
            jax.ShapeDtypeStruct((N_NODES, dout), jnp.float32),
        ],
    )(parts, r_prev, w_rel, w_root, b.reshape(1, dout))


def _tc_mid_h(parts, r_prev, w_root, b):
    din = r_prev.shape[1]
    dout = w_root.shape[1]

    def body(p_ref, rp_ref, wt_ref, b_ref, h_ref, r_ref):
        h = p_ref[0] + p_ref[1] + rp_ref[...]
        h = h * jax.nn.sigmoid(h)
        h_ref[...] = h
        r_ref[...] = (
            jnp.dot(h, wt_ref[...], preferred_element_type=jnp.float32)
            + b_ref[...]
        )

    grid = (N_NODES // ROW_BLK,)
    return pl.pallas_call(
        body,
        grid=grid,
        in_specs=[
            pl.BlockSpec((2, ROW_BLK, din), lambda i: (0, i, 0)),
            pl.BlockSpec((ROW_BLK, din), lambda i: (i, 0)),
            pl.BlockSpec(w_root.shape, lambda i: (0, 0)),
            pl.BlockSpec((1, dout), lambda i: (0, 0)),
        ],
        out_specs=[
            pl.BlockSpec((ROW_BLK, din), lambda i: (i, 0)),
            pl.BlockSpec((ROW_BLK, dout), lambda i: (i, 0)),
        ],
        out_shape=[
            jax.ShapeDtypeStruct((N_NODES, din), jnp.float32),
            jax.ShapeDtypeStruct((N_NODES, dout), jnp.float32),
        ],
    )(parts, r_prev, w_root, b.reshape(1, dout))


def _tc_final(parts, r_prev, w_rel):
    din = w_rel.shape[0]
    d = r_prev.shape[1]

    def body(p_ref, rp_ref, wr_ref, o_ref):
        g = p_ref[0] + p_ref[1]
        h = (
            jnp.dot(g, wr_ref[...], preferred_element_type=jnp.float32)
            + rp_ref[...]
        )
        m = jnp.max(h, axis=1, keepdims=True)
        s = h - m
        o_ref[...] = s - jnp.log(jnp.sum(jnp.exp(s), axis=1, keepdims=True))

    grid = (N_NODES // ROW_BLK,)
    return pl.pallas_call(
        body,
        grid=grid,
        in_specs=[
            pl.BlockSpec((2, ROW_BLK, din), lambda i: (0, i, 0)),
            pl.BlockSpec((ROW_BLK, d), lambda i: (i, 0)),
            pl.BlockSpec(w_rel.shape, lambda i: (0, 0)),
        ],
        out_specs=pl.BlockSpec((ROW_BLK, d), lambda i: (i, 0)),
        out_shape=jax.ShapeDtypeStruct((N_NODES, d), jnp.float32),
    )(parts, r_prev, w_rel)


_seg_sum_128 = _make_seg_sum(128)


def kernel(x, edge_index, W_rel1, b_rel1, W_root1, W_rel2, b_rel2, W_root2,
           W_rel3, b_rel3, W_root3):
    pad = E_PAD - N_EDGES
    src = jnp.concatenate(
        [edge_index[0].astype(jnp.int32),
         jnp.arange(pad, dtype=jnp.int32) % N_NODES]
    ).reshape(E_PAD // CHUNK, CHUNK)
    dst = jnp.concatenate(
        [edge_index[1].astype(jnp.int32),
         N_NODES + jnp.arange(pad, dtype=jnp.int32) % (ACC_ROWS - N_NODES)]
    ).reshape(E_PAD // CHUNK, CHUNK)

    y1, r1 = _tc_first(x, W_rel1, W_root1, b_rel1)
    p1 = _seg_sum_128(y1, src, dst)
    y2, r2 = _tc_mid(p1, r1, W_rel2, W_root2, b_rel2)
    p2 = _seg_sum_128(y2, src, dst)
    h2, r3 = _tc_mid_h(p2, r2, W_root3, b_rel3)
    p3 = _seg_sum_128(h2, src, dst)
    return _tc_final(p3, r3, W_rel3)
